# baseline (speedup 1.0000x reference)
import jax
import jax.numpy as jnp
from jax import lax
from jax.experimental import pallas as pl
from jax.experimental.pallas import tpu as pltpu
from jax.experimental.pallas import tpu_sc as plsc

N_NODES = 10000
N_EDGES = 160000
N_PRED = 100000
IN_CH = 980
HID = 256
OUT = 64
PRED_HID = 128

NC, NS = 2, 16
MP = 10240
KP = 1024
EC = 128
ECH = 80
E_PAD = NS * ECH * EC
PCH = 26
P_PAD = NC * NS * PCH * 128
MB = 256

_SC_PARAMS = pltpu.CompilerParams(needs_layout_passes=False)


def _dual_mm(xt, Wa, Wb):
    K = xt.shape[0]
    Fo = Wa.shape[1]
    dn = (((0,), (0,)), ((), ()))

    def body(x_ref, wa_ref, wb_ref, star_ref, xr_ref):
        xb = x_ref[...]
        ol = lax.dot_general(xb, wa_ref[...], dn,
                             preferred_element_type=jnp.float32)
        star_ref[...] = jnp.stack([ol[:, :128], ol[:, 128:]])
        xr_ref[...] = lax.dot_general(xb, wb_ref[...], dn,
                                      preferred_element_type=jnp.float32)

    return pl.pallas_call(
        body,
        grid=(MP // MB,),
        in_specs=[
            pl.BlockSpec((K, MB), lambda m: (0, m)),
            pl.BlockSpec((K, Fo), lambda m: (0, 0)),
            pl.BlockSpec((K, Fo), lambda m: (0, 0)),
        ],
        out_specs=[
            pl.BlockSpec((2, MB, 128), lambda m: (0, m, 0)),
            pl.BlockSpec((MB, Fo), lambda m: (m, 0)),
        ],
        out_shape=[
            jax.ShapeDtypeStruct((2, MP, 128), jnp.float32),
            jax.ShapeDtypeStruct((MP, Fo), jnp.float32),
        ],
    )(xt, Wa, Wb)


def _tc_stage(s_star, cntp, xr, b, Wa, Wb, relu, split):
    Fin = xr.shape[1]
    Fo = Wa.shape[1]

    def body(s0_ref, s1_ref, cnt_ref, xr_ref, b_ref, wa_ref, wb_ref,
             star_ref, xro_ref):
        s = jnp.concatenate([s0_ref[0], s1_ref[0]], axis=-1)
        cnt = jnp.sum(cnt_ref[...], axis=0)[:, None]
        inv = 1.0 / jnp.maximum(cnt, 1.0)
        h = s * inv + b_ref[...] + xr_ref[...]
        if relu:
            h = jnp.maximum(h, 0.0)
        ol = jnp.dot(h, wa_ref[...], preferred_element_type=jnp.float32)
        if split:
            star_ref[...] = jnp.stack([ol[:, :128], ol[:, 128:]])
        else:
            olp = jnp.concatenate(
                [ol, jnp.zeros((MB, 128 - Fo), jnp.float32)], axis=-1)
            star_ref[...] = jnp.stack([olp, olp])
        xro_ref[...] = jnp.dot(h, wb_ref[...], preferred_element_type=jnp.float32)

    star_spec = pl.BlockSpec((2, MB, 128), lambda m: (0, m, 0))
    star_shape = (2, MP, 128)
    return pl.pallas_call(
        body,
        grid=(MP // MB,),
        in_specs=[
            pl.BlockSpec((1, MB, 128), lambda m: (0, m, 0)),
            pl.BlockSpec((1, MB, 128), lambda m: (1, m, 0)),
            pl.BlockSpec((NC * NS, MB), lambda m: (0, m)),
            pl.BlockSpec((MB, Fin), lambda m: (m, 0)),
            pl.BlockSpec((1, Fin), lambda m: (0, 0)),
            pl.BlockSpec((Fin, Fo), lambda m: (0, 0)),
            pl.BlockSpec((Fin, Fo), lambda m: (0, 0)),
        ],
        out_specs=[
            star_spec,
            pl.BlockSpec((MB, Fo), lambda m: (m, 0)),
        ],
        out_shape=[
            jax.ShapeDtypeStruct(star_shape, jnp.float32),
            jax.ShapeDtypeStruct((MP, Fo), jnp.float32),
        ],
    )(s_star, s_star, cntp, xr, b, Wa, Wb)


def _tc_final_nodes(s3p, cntp, xr, b, Wp1, bp1):
    Fin = xr.shape[1]

    def body(s0_ref, s1_ref, cnt_ref, xr_ref, b_ref, w_ref, bp1_ref,
             star_ref):
        z = (s0_ref[0] + s1_ref[0])[:, :Fin]
        cnt = jnp.sum(cnt_ref[...], axis=0)[:, None]
        inv = 1.0 / jnp.maximum(cnt, 1.0)
        z = z * inv + b_ref[...] + xr_ref[...]
        w = w_ref[...]
        a = jnp.dot(z, w[:Fin], preferred_element_type=jnp.float32)
        a = a + bp1_ref[...]
        bz = jnp.dot(z, w[Fin:], preferred_element_type=jnp.float32)
        star_ref[...] = jnp.stack([a, bz, a, bz])

    return pl.pallas_call(
        body,
        grid=(MP // MB,),
        in_specs=[
            pl.BlockSpec((1, MB, 128), lambda m: (0, m, 0)),
            pl.BlockSpec((1, MB, 128), lambda m: (1, m, 0)),
            pl.BlockSpec((NC * NS, MB), lambda m: (0, m)),
            pl.BlockSpec((MB, Fin), lambda m: (m, 0)),
            pl.BlockSpec((1, Fin), lambda m: (0, 0)),
            pl.BlockSpec((2 * Fin, PRED_HID), lambda m: (0, 0)),
            pl.BlockSpec((1, PRED_HID), lambda m: (0, 0)),
        ],
        out_specs=pl.BlockSpec((4, MB, PRED_HID), lambda m: (0, m, 0)),
        out_shape=jax.ShapeDtypeStruct((4, MP, PRED_HID), jnp.float32),
    )(s3p, s3p, cntp, xr, b, Wp1, bp1)


def _degree(dst_slab):
    half = ECH // 2
    mesh = plsc.VectorSubcoreMesh(core_axis_name="c", subcore_axis_name="s")
    scratch = [
        pltpu.VMEM((MP,), jnp.float32),
        pltpu.VMEM((1, EC), jnp.int32),
        pltpu.VMEM((1, EC), jnp.int32),
        pltpu.SemaphoreType.DMA,
        pltpu.SemaphoreType.DMA,
    ]

    def body(dst_ref, out_ref, cntp, id0, id1, sd0, sd1):
        c = lax.axis_index("c")
        s = lax.axis_index("s")
        base = c * half

        def czero(i, carry):
            cntp[pl.ds(i * 16, 16)] = jnp.zeros((16,), jnp.float32)
            return carry
        lax.fori_loop(0, MP // 16, czero, None)

        pltpu.sync_copy(dst_ref.at[s, base], id0.at[0])
        pltpu.sync_copy(dst_ref.at[s, base + 1], id1.at[0])

        ones16 = jnp.full((16,), 1.0, jnp.float32)

        def step(i, carry):
            for par, idb, sd in ((0, id0, sd0), (1, id1, sd1)):
                j = 2 * i + par

                @pl.when(j >= 2)
                def _():
                    pltpu.make_async_copy(dst_ref.at[s, base + j], idb.at[0],
                                          sd).wait()

                for k in range(EC // 16):
                    idv = idb[0, pl.ds(k * 16, 16)]
                    plsc.addupdate_scatter(cntp, [idv], ones16)

                @pl.when(j + 2 < half)
                def _():
                    pltpu.async_copy(dst_ref.at[s, base + j + 2], idb.at[0],
                                     sd)
            return carry
        lax.fori_loop(0, half // 2, step, None)

        pltpu.sync_copy(cntp, out_ref.at[c, s])

    fn = pl.kernel(body,
                   out_type=jax.ShapeDtypeStruct((NC, NS, MP), jnp.float32),
                   mesh=mesh, scratch_types=scratch,
                   compiler_params=_SC_PARAMS)
    return fn(dst_slab)


def _seg_half(table, src_adj, dst_slab):
    rpt = MP // NS

    mesh = plsc.VectorSubcoreMesh(core_axis_name="c", subcore_axis_name="s")
    scratch = [
        pltpu.VMEM_SHARED((MP, 128), jnp.float32),
        pltpu.VMEM((EC,), jnp.int32),
        pltpu.VMEM((EC,), jnp.int32),
        pltpu.VMEM((1, EC), jnp.int32),
        pltpu.VMEM((1, EC), jnp.int32),
        pltpu.VMEM((EC, 128), jnp.float32),
        pltpu.VMEM((EC, 128), jnp.float32),
        pltpu.SemaphoreType.DMA,
        pltpu.SemaphoreType.DMA,
        pltpu.SemaphoreType.DMA,
        pltpu.SemaphoreType.DMA,
        pltpu.SemaphoreType.DMA,
        pltpu.SemaphoreType.DMA,
    ]

    def body(table_ref, src_ref, dst_ref, out_ref,
             acc, is0, is1, id0, id1, buf0, buf1,
             sem0, sem1, sis0, sis1, sid0, sid1):
        c = lax.axis_index("c")
        s = lax.axis_index("s")

        def zfill(i, carry):
            for k in range(8):
                buf0[i, pl.ds(k * 16, 16)] = jnp.zeros((16,), jnp.float32)
            return carry
        lax.fori_loop(0, EC, zfill, None)
        for t in range(rpt // EC):
            pltpu.sync_copy(buf0, acc.at[pl.ds(s * rpt + t * EC, EC)])

        plsc.subcore_barrier()

        pltpu.sync_copy(src_ref.at[c, s, 0], is0)
        pltpu.sync_copy(dst_ref.at[s, 0], id0.at[0])
        pltpu.sync_copy(src_ref.at[c, s, 1], is1)
        pltpu.sync_copy(dst_ref.at[s, 1], id1.at[0])
        pltpu.async_copy(table_ref.at[is0], buf0, sem0)
        pltpu.async_copy(table_ref.at[is1], buf1, sem1)

        def step(i, carry):
            for par, isb, idb, buf, sem, sis, sid in (
                    (0, is0, id0, buf0, sem0, sis0, sid0),
                    (1, is1, id1, buf1, sem1, sis1, sid1)):
                j = 2 * i + par
                pltpu.make_async_copy(table_ref.at[isb], buf, sem).wait()

                @pl.when(j + 2 < ECH)
                def _():
                    pltpu.async_copy(src_ref.at[c, s, j + 2], isb, sis)

                @pl.when(j >= 2)
                def _():
                    pltpu.make_async_copy(dst_ref.at[s, j], idb.at[0],
                                          sid).wait()

                pltpu.sync_copy(buf, acc.at[idb.at[0]], add=True)

                @pl.when(j + 2 < ECH)
                def _():
                    pltpu.async_copy(dst_ref.at[s, j + 2], idb.at[0], sid)
                    pltpu.make_async_copy(src_ref.at[c, s, j + 2], isb,
                                          sis).wait()
                    pltpu.async_copy(table_ref.at[isb], buf, sem)
            return carry
        lax.fori_loop(0, ECH // 2, step, None)

        plsc.subcore_barrier()
        pltpu.sync_copy(acc.at[pl.ds(s * rpt, rpt)],
                        out_ref.at[c, pl.ds(s * rpt, rpt)])

    fn = pl.kernel(body,
                   out_type=jax.ShapeDtypeStruct((2, MP, 128), jnp.float32),
                   mesh=mesh, scratch_types=scratch,
                   compiler_params=_SC_PARAMS)
    return fn(table, src_adj, dst_slab)


def _seg_ep(table, src_adj, dst_slab):
    rpt = MP // NS
    half = ECH // 2

    mesh = plsc.VectorSubcoreMesh(core_axis_name="c", subcore_axis_name="s")
    scratch = [
        pltpu.VMEM_SHARED((MP, 128), jnp.float32),
        pltpu.VMEM((EC,), jnp.int32),
        pltpu.VMEM((EC,), jnp.int32),
        pltpu.VMEM((1, EC), jnp.int32),
        pltpu.VMEM((1, EC), jnp.int32),
        pltpu.VMEM((EC, 128), jnp.float32),
        pltpu.VMEM((EC, 128), jnp.float32),
        pltpu.SemaphoreType.DMA,
        pltpu.SemaphoreType.DMA,
        pltpu.SemaphoreType.DMA,
        pltpu.SemaphoreType.DMA,
        pltpu.SemaphoreType.DMA,
        pltpu.SemaphoreType.DMA,
    ]

    def body(table_ref, src_ref, dst_ref, out_ref,
             acc, is0, is1, id0, id1, buf0, buf1,
             sem0, sem1, sis0, sis1, sid0, sid1):
        c = lax.axis_index("c")
        s = lax.axis_index("s")
        base = c * half

        def zfill(i, carry):
            for k in range(8):
                buf0[i, pl.ds(k * 16, 16)] = jnp.zeros((16,), jnp.float32)
            return carry
        lax.fori_loop(0, EC, zfill, None)
        for t in range(rpt // EC):
            pltpu.sync_copy(buf0, acc.at[pl.ds(s * rpt + t * EC, EC)])

        plsc.subcore_barrier()

        pltpu.sync_copy(src_ref.at[c, s, base], is0)
        pltpu.sync_copy(dst_ref.at[s, base], id0.at[0])
        pltpu.sync_copy(src_ref.at[c, s, base + 1], is1)
        pltpu.sync_copy(dst_ref.at[s, base + 1], id1.at[0])
        pltpu.async_copy(table_ref.at[is0], buf0, sem0)
        pltpu.async_copy(table_ref.at[is1], buf1, sem1)

        def step(i, carry):
            for par, isb, idb, buf, sem, sis, sid in (
                    (0, is0, id0, buf0, sem0, sis0, sid0),
                    (1, is1, id1, buf1, sem1, sis1, sid1)):
                j = 2 * i + par
                g = base + j
                pltpu.make_async_copy(table_ref.at[isb], buf, sem).wait()

                @pl.when(j + 2 < half)
                def _():
                    pltpu.async_copy(src_ref.at[c, s, g + 2], isb, sis)

                @pl.when(j >= 2)
                def _():
                    pltpu.make_async_copy(dst_ref.at[s, g], idb.at[0],
                                          sid).wait()

                pltpu.sync_copy(buf, acc.at[idb.at[0]], add=True)

                @pl.when(j + 2 < half)
                def _():
                    pltpu.async_copy(dst_ref.at[s, g + 2], idb.at[0], sid)
                    pltpu.make_async_copy(src_ref.at[c, s, g + 2], isb,
                                          sis).wait()
                    pltpu.async_copy(table_ref.at[isb], buf, sem)
            return carry
        lax.fori_loop(0, half // 2, step, None)

        plsc.subcore_barrier()
        pltpu.sync_copy(acc.at[pl.ds(s * rpt, rpt)],
                        out_ref.at[c, pl.ds(s * rpt, rpt)])

    fn = pl.kernel(
        body,
        out_type=jax.ShapeDtypeStruct((2, MP, 128), jnp.float32),
        mesh=mesh,
        scratch_types=scratch,
        compiler_params=_SC_PARAMS,
    )
    return fn(table, src_adj, dst_slab)


def _pred_fused(table, pidx, wp2, bp2p):
    mesh = plsc.VectorSubcoreMesh(core_axis_name="c", subcore_axis_name="s")
    scratch = [
        pltpu.VMEM((PCH, 128), jnp.int32),
        pltpu.VMEM((PCH, 128), jnp.int32),
        pltpu.VMEM((128, 128), jnp.float32),
        pltpu.VMEM((128, 128), jnp.float32),
        pltpu.VMEM((128, 128), jnp.float32),
        pltpu.VMEM((128, 128), jnp.float32),
        pltpu.VMEM((128,), jnp.float32),
        pltpu.VMEM((16 * 17,), jnp.float32),
        pltpu.VMEM((PRED_HID,), jnp.float32),
        pltpu.VMEM((16,), jnp.float32),
        pltpu.SemaphoreType.DMA,
        pltpu.SemaphoreType.DMA,
        pltpu.SemaphoreType.DMA,
        pltpu.SemaphoreType.DMA,
    ]
    NK = PRED_HID // 16

    def body(table_ref, pidx_ref, wp2_ref, bp2_ref, out_ref,
             idx_a, idx_b, bufa0, bufa1, bufb0, bufb1, lbuf, tbuf,
             cwp2, cbp2, sa0, sa1, sb0, sb1):
        c = lax.axis_index("c")
        s = lax.axis_index("s")
        w = c * NS + s
        base = w * (PCH * 128)
        pltpu.sync_copy(pidx_ref.at[0, w], idx_a)
        pltpu.sync_copy(pidx_ref.at[1, w], idx_b)
        pltpu.sync_copy(wp2_ref, cwp2)
        pltpu.sync_copy(bp2_ref, cbp2)

        pltpu.async_copy(table_ref.at[idx_a.at[0]], bufa0, sa0)
        pltpu.async_copy(table_ref.at[idx_b.at[0]], bufb0, sb0)
        pltpu.async_copy(table_ref.at[idx_a.at[1]], bufa1, sa1)
        pltpu.async_copy(table_ref.at[idx_b.at[1]], bufb1, sb1)

        wp2c = [cwp2[pl.ds(k * 16, 16)] for k in range(NK)]
        bp2v = cbp2[pl.ds(0, 16)]
        lane17 = lax.iota(jnp.int32, 16) * 17

        def step(i, carry):
            for par, bufa, bufb, sa, sb in ((0, bufa0, bufb0, sa0, sb0),
                                            (1, bufa1, bufb1, sa1, sb1)):
                j = 2 * i + par
                pltpu.make_async_copy(table_ref.at[idx_a.at[j]], bufa,
                                      sa).wait()
                pltpu.make_async_copy(table_ref.at[idx_b.at[j]], bufb,
                                      sb).wait()

                def group(g, carry2):
                    def edge(ee, carry3):
                        e = g * 16 + ee
                        acc = jnp.zeros((16,), jnp.float32)
                        for k in range(NK):
                            va = bufa[e, pl.ds(k * 16, 16)]
                            vb = bufb[e, pl.ds(k * 16, 16)]
                            acc = acc + jnp.maximum(va + vb, 0.0) * wp2c[k]
                        plsc.store_scatter(tbuf, [lane17 + ee], acc)
                        return carry3
                    lax.fori_loop(0, 16, edge, None)

                    v = tbuf[pl.ds(0, 16)]
                    for l in range(1, 16):
                        v = v + tbuf[pl.ds(l * 17, 16)]
                    v = v + bp2v
                    lbuf[pl.ds(g * 16, 16)] = 1.0 / (1.0 + jnp.exp(-v))
                    return carry2
                lax.fori_loop(0, 8, group, None)

                pltpu.sync_copy(lbuf, out_ref.at[pl.ds(base + j * 128, 128)])

                @pl.when(j + 2 < PCH)
                def _():
                    pltpu.async_copy(table_ref.at[idx_a.at[j + 2]], bufa, sa)
                    pltpu.async_copy(table_ref.at[idx_b.at[j + 2]], bufb, sb)
            return carry
        lax.fori_loop(0, PCH // 2, step, None)

    fn = pl.kernel(
        body,
        out_type=jax.ShapeDtypeStruct((P_PAD,), jnp.float32),
        mesh=mesh,
        scratch_types=scratch,
        compiler_params=_SC_PARAMS,
    )
    return fn(table, pidx, wp2, bp2p)


def kernel(x, edge_index, pred_edge_index, W1l, b1l, W1r, W2l, b2l, W2r,
           W3l, b3l, W3r, Wp1, bp1, Wp2, bp2):
    ei = edge_index.astype(jnp.int32)
    pei = pred_edge_index.astype(jnp.int32)

    src = jnp.pad(ei[0], (0, E_PAD - N_EDGES))
    dst = jnp.pad(ei[1], (0, E_PAD - N_EDGES), constant_values=N_NODES)
    src_slab = src.reshape(NS, ECH, EC)
    src_adj = jnp.stack([src_slab, src_slab + MP])
    dst_slab = dst.reshape(NS, ECH, EC)

    ps = jnp.pad(pei[0], (0, P_PAD - N_PRED)).reshape(NC * NS, PCH, 128)
    pd = jnp.pad(pei[1], (0, P_PAD - N_PRED)).reshape(NC * NS, PCH, 128)
    woff = (jnp.arange(NC * NS, dtype=jnp.int32) // NS * (2 * MP))[:, None, None]
    pidx = jnp.stack([ps + woff, pd + MP + woff])

    cntp = _degree(dst_slab).reshape(NC * NS, MP)

    star1, xr1 = _dual_mm(x.T, W1l, W1r)
    s1 = _seg_half(star1.reshape(2 * MP, 128), src_adj, dst_slab)

    star2, xr2 = _tc_stage(s1, cntp, xr1, b1l.reshape(1, HID), W2l, W2r,
                           relu=True, split=True)
    s2 = _seg_half(star2.reshape(2 * MP, 128), src_adj, dst_slab)
    star3, xr3 = _tc_stage(s2, cntp, xr2, b2l.reshape(1, HID), W3l, W3r,
                           relu=True, split=False)
    s3p = _seg_ep(star3.reshape(2 * MP, 128), src_adj, dst_slab)

    starp = _tc_final_nodes(s3p, cntp, xr3, b3l.reshape(1, OUT), Wp1,
                            bp1.reshape(1, PRED_HID))
    out = _pred_fused(starp.reshape(4 * MP, PRED_HID), pidx,
                      Wp2.reshape(PRED_HID), jnp.broadcast_to(bp2, (16,)))
    return out[:N_PRED]


# speedup vs baseline: 1.1030x; 1.1030x over previous
import jax
import jax.numpy as jnp
from jax import lax
from jax.experimental import pallas as pl
from jax.experimental.pallas import tpu as pltpu
from jax.experimental.pallas import tpu_sc as plsc

N_NODES = 10000
N_EDGES = 160000
N_PRED = 100000
IN_CH = 980
HID = 256
OUT = 64
PRED_HID = 128

NC, NS = 2, 16
MP = 10240
KP = 1024
EC = 128
ECH = 80
E_PAD = NS * ECH * EC
PCH = 26
P_PAD = NC * NS * PCH * 128
MB = 256

_SC_PARAMS = pltpu.CompilerParams(needs_layout_passes=False)


def _dual_mm(xt, Wa, Wb):
    K = xt.shape[0]
    Fo = Wa.shape[1]
    dn = (((0,), (0,)), ((), ()))

    def body(x_ref, wa_ref, wb_ref, star_ref, xr_ref):
        xb = x_ref[...]
        ol = lax.dot_general(xb, wa_ref[...], dn,
                             preferred_element_type=jnp.float32)
        star_ref[...] = jnp.stack([ol[:, :128], ol[:, 128:]])
        xr_ref[...] = lax.dot_general(xb, wb_ref[...], dn,
                                      preferred_element_type=jnp.float32)

    return pl.pallas_call(
        body,
        grid=(MP // MB,),
        in_specs=[
            pl.BlockSpec((K, MB), lambda m: (0, m)),
            pl.BlockSpec((K, Fo), lambda m: (0, 0)),
            pl.BlockSpec((K, Fo), lambda m: (0, 0)),
        ],
        out_specs=[
            pl.BlockSpec((2, MB, 128), lambda m: (0, m, 0)),
            pl.BlockSpec((MB, Fo), lambda m: (m, 0)),
        ],
        out_shape=[
            jax.ShapeDtypeStruct((2, MP, 128), jnp.float32),
            jax.ShapeDtypeStruct((MP, Fo), jnp.float32),
        ],
    )(xt, Wa, Wb)


def _tc_stage(s_star, cntp, xr, b, Wa, Wb, relu, split):
    Fin = xr.shape[1]
    Fo = Wa.shape[1]

    def body(s0_ref, s1_ref, cnt_ref, xr_ref, b_ref, wa_ref, wb_ref,
             star_ref, xro_ref):
        s = jnp.concatenate([s0_ref[0], s1_ref[0]], axis=-1)
        cnt = jnp.sum(cnt_ref[...], axis=0)[:, None]
        inv = 1.0 / jnp.maximum(cnt, 1.0)
        h = s * inv + b_ref[...] + xr_ref[...]
        if relu:
            h = jnp.maximum(h, 0.0)
        ol = jnp.dot(h, wa_ref[...], preferred_element_type=jnp.float32)
        if split:
            star_ref[...] = jnp.stack([ol[:, :128], ol[:, 128:]])
        else:
            star_ref[...] = jnp.concatenate(
                [ol, jnp.zeros((MB, 128 - Fo), jnp.float32)], axis=-1)
        xro_ref[...] = jnp.dot(h, wb_ref[...], preferred_element_type=jnp.float32)

    star_spec = (pl.BlockSpec((2, MB, 128), lambda m: (0, m, 0)) if split
                 else pl.BlockSpec((MB, 128), lambda m: (m, 0)))
    star_shape = ((2, MP, 128) if split else (MP, 128))
    return pl.pallas_call(
        body,
        grid=(MP // MB,),
        in_specs=[
            pl.BlockSpec((1, MB, 128), lambda m: (0, m, 0)),
            pl.BlockSpec((1, MB, 128), lambda m: (1, m, 0)),
            pl.BlockSpec((NC * NS, MB), lambda m: (0, m)),
            pl.BlockSpec((MB, Fin), lambda m: (m, 0)),
            pl.BlockSpec((1, Fin), lambda m: (0, 0)),
            pl.BlockSpec((Fin, Fo), lambda m: (0, 0)),
            pl.BlockSpec((Fin, Fo), lambda m: (0, 0)),
        ],
        out_specs=[
            star_spec,
            pl.BlockSpec((MB, Fo), lambda m: (m, 0)),
        ],
        out_shape=[
            jax.ShapeDtypeStruct(star_shape, jnp.float32),
            jax.ShapeDtypeStruct((MP, Fo), jnp.float32),
        ],
    )(s_star, s_star, cntp, xr, b, Wa, Wb)


def _tc_final_nodes(s3p, cntp, xr, b, Wp1, bp1):
    Fin = xr.shape[1]

    def body(s0_ref, s1_ref, cnt_ref, xr_ref, b_ref, w_ref, bp1_ref,
             star_ref):
        z = (s0_ref[0] + s1_ref[0])[:, :Fin]
        cnt = jnp.sum(cnt_ref[...], axis=0)[:, None]
        inv = 1.0 / jnp.maximum(cnt, 1.0)
        z = z * inv + b_ref[...] + xr_ref[...]
        w = w_ref[...]
        a = jnp.dot(z, w[:Fin], preferred_element_type=jnp.float32)
        a = a + bp1_ref[...]
        bz = jnp.dot(z, w[Fin:], preferred_element_type=jnp.float32)
        star_ref[...] = jnp.stack([a, bz])

    return pl.pallas_call(
        body,
        grid=(MP // MB,),
        in_specs=[
            pl.BlockSpec((1, MB, 128), lambda m: (0, m, 0)),
            pl.BlockSpec((1, MB, 128), lambda m: (1, m, 0)),
            pl.BlockSpec((NC * NS, MB), lambda m: (0, m)),
            pl.BlockSpec((MB, Fin), lambda m: (m, 0)),
            pl.BlockSpec((1, Fin), lambda m: (0, 0)),
            pl.BlockSpec((2 * Fin, PRED_HID), lambda m: (0, 0)),
            pl.BlockSpec((1, PRED_HID), lambda m: (0, 0)),
        ],
        out_specs=pl.BlockSpec((2, MB, PRED_HID), lambda m: (0, m, 0)),
        out_shape=jax.ShapeDtypeStruct((2, MP, PRED_HID), jnp.float32),
    )(s3p, s3p, cntp, xr, b, Wp1, bp1)


def _degree(dst_slab):
    half = ECH // 2
    mesh = plsc.VectorSubcoreMesh(core_axis_name="c", subcore_axis_name="s")
    scratch = [
        pltpu.VMEM((MP,), jnp.float32),
        pltpu.VMEM((1, EC), jnp.int32),
        pltpu.VMEM((1, EC), jnp.int32),
        pltpu.SemaphoreType.DMA,
        pltpu.SemaphoreType.DMA,
    ]

    def body(dst_ref, out_ref, cntp, id0, id1, sd0, sd1):
        c = lax.axis_index("c")
        s = lax.axis_index("s")
        base = c * half

        def czero(i, carry):
            cntp[pl.ds(i * 16, 16)] = jnp.zeros((16,), jnp.float32)
            return carry
        lax.fori_loop(0, MP // 16, czero, None)

        pltpu.sync_copy(dst_ref.at[s, base], id0.at[0])
        pltpu.sync_copy(dst_ref.at[s, base + 1], id1.at[0])

        ones16 = jnp.full((16,), 1.0, jnp.float32)

        def step(i, carry):
            for par, idb, sd in ((0, id0, sd0), (1, id1, sd1)):
                j = 2 * i + par

                @pl.when(j >= 2)
                def _():
                    pltpu.make_async_copy(dst_ref.at[s, base + j], idb.at[0],
                                          sd).wait()

                for k in range(EC // 16):
                    idv = idb[0, pl.ds(k * 16, 16)]
                    plsc.addupdate_scatter(cntp, [idv], ones16)

                @pl.when(j + 2 < half)
                def _():
                    pltpu.async_copy(dst_ref.at[s, base + j + 2], idb.at[0],
                                     sd)
            return carry
        lax.fori_loop(0, half // 2, step, None)

        pltpu.sync_copy(cntp, out_ref.at[c, s])

    fn = pl.kernel(body,
                   out_type=jax.ShapeDtypeStruct((NC, NS, MP), jnp.float32),
                   mesh=mesh, scratch_types=scratch,
                   compiler_params=_SC_PARAMS)
    return fn(dst_slab)


def _seg_half(table, src_adj, dst_slab):
    rpt = MP // NS

    mesh = plsc.VectorSubcoreMesh(core_axis_name="c", subcore_axis_name="s")
    scratch = [
        pltpu.VMEM_SHARED((MP, 128), jnp.float32),
        pltpu.VMEM((EC,), jnp.int32),
        pltpu.VMEM((EC,), jnp.int32),
        pltpu.VMEM((1, EC), jnp.int32),
        pltpu.VMEM((1, EC), jnp.int32),
        pltpu.VMEM((EC, 128), jnp.float32),
        pltpu.VMEM((EC, 128), jnp.float32),
        pltpu.SemaphoreType.DMA,
        pltpu.SemaphoreType.DMA,
        pltpu.SemaphoreType.DMA,
        pltpu.SemaphoreType.DMA,
        pltpu.SemaphoreType.DMA,
        pltpu.SemaphoreType.DMA,
    ]

    def body(table_ref, src_ref, dst_ref, out_ref,
             acc, is0, is1, id0, id1, buf0, buf1,
             sem0, sem1, sis0, sis1, sid0, sid1):
        c = lax.axis_index("c")
        s = lax.axis_index("s")

        def zfill(i, carry):
            for k in range(8):
                buf0[i, pl.ds(k * 16, 16)] = jnp.zeros((16,), jnp.float32)
            return carry
        lax.fori_loop(0, EC, zfill, None)
        for t in range(rpt // EC):
            pltpu.sync_copy(buf0, acc.at[pl.ds(s * rpt + t * EC, EC)])

        plsc.subcore_barrier()

        pltpu.sync_copy(src_ref.at[c, s, 0], is0)
        pltpu.sync_copy(dst_ref.at[s, 0], id0.at[0])
        pltpu.sync_copy(src_ref.at[c, s, 1], is1)
        pltpu.sync_copy(dst_ref.at[s, 1], id1.at[0])
        pltpu.async_copy(table_ref.at[is0], buf0, sem0)
        pltpu.async_copy(table_ref.at[is1], buf1, sem1)

        def step(i, carry):
            for par, isb, idb, buf, sem, sis, sid in (
                    (0, is0, id0, buf0, sem0, sis0, sid0),
                    (1, is1, id1, buf1, sem1, sis1, sid1)):
                j = 2 * i + par
                pltpu.make_async_copy(table_ref.at[isb], buf, sem).wait()

                @pl.when(j + 2 < ECH)
                def _():
                    pltpu.async_copy(src_ref.at[c, s, j + 2], isb, sis)

                @pl.when(j >= 2)
                def _():
                    pltpu.make_async_copy(dst_ref.at[s, j], idb.at[0],
                                          sid).wait()

                pltpu.sync_copy(buf, acc.at[idb.at[0]], add=True)

                @pl.when(j + 2 < ECH)
                def _():
                    pltpu.async_copy(dst_ref.at[s, j + 2], idb.at[0], sid)
                    pltpu.make_async_copy(src_ref.at[c, s, j + 2], isb,
                                          sis).wait()
                    pltpu.async_copy(table_ref.at[isb], buf, sem)
            return carry
        lax.fori_loop(0, ECH // 2, step, None)

        plsc.subcore_barrier()
        pltpu.sync_copy(acc.at[pl.ds(s * rpt, rpt)],
                        out_ref.at[c, pl.ds(s * rpt, rpt)])

    fn = pl.kernel(body,
                   out_type=jax.ShapeDtypeStruct((2, MP, 128), jnp.float32),
                   mesh=mesh, scratch_types=scratch,
                   compiler_params=_SC_PARAMS)
    return fn(table, src_adj, dst_slab)


def _seg_ep(table, src_adj, dst_slab):
    rpt = MP // NS
    half = ECH // 2

    mesh = plsc.VectorSubcoreMesh(core_axis_name="c", subcore_axis_name="s")
    scratch = [
        pltpu.VMEM_SHARED((MP, 128), jnp.float32),
        pltpu.VMEM((EC,), jnp.int32),
        pltpu.VMEM((EC,), jnp.int32),
        pltpu.VMEM((1, EC), jnp.int32),
        pltpu.VMEM((1, EC), jnp.int32),
        pltpu.VMEM((EC, 128), jnp.float32),
        pltpu.VMEM((EC, 128), jnp.float32),
        pltpu.SemaphoreType.DMA,
        pltpu.SemaphoreType.DMA,
        pltpu.SemaphoreType.DMA,
        pltpu.SemaphoreType.DMA,
        pltpu.SemaphoreType.DMA,
        pltpu.SemaphoreType.DMA,
    ]

    def body(table_ref, src_ref, dst_ref, out_ref,
             acc, is0, is1, id0, id1, buf0, buf1,
             sem0, sem1, sis0, sis1, sid0, sid1):
        c = lax.axis_index("c")
        s = lax.axis_index("s")
        base = c * half

        def zfill(i, carry):
            for k in range(8):
                buf0[i, pl.ds(k * 16, 16)] = jnp.zeros((16,), jnp.float32)
            return carry
        lax.fori_loop(0, EC, zfill, None)
        for t in range(rpt // EC):
            pltpu.sync_copy(buf0, acc.at[pl.ds(s * rpt + t * EC, EC)])

        plsc.subcore_barrier()

        pltpu.sync_copy(src_ref.at[0, s, base], is0)
        pltpu.sync_copy(dst_ref.at[s, base], id0.at[0])
        pltpu.sync_copy(src_ref.at[0, s, base + 1], is1)
        pltpu.sync_copy(dst_ref.at[s, base + 1], id1.at[0])
        pltpu.async_copy(table_ref.at[is0], buf0, sem0)
        pltpu.async_copy(table_ref.at[is1], buf1, sem1)

        def step(i, carry):
            for par, isb, idb, buf, sem, sis, sid in (
                    (0, is0, id0, buf0, sem0, sis0, sid0),
                    (1, is1, id1, buf1, sem1, sis1, sid1)):
                j = 2 * i + par
                g = base + j
                pltpu.make_async_copy(table_ref.at[isb], buf, sem).wait()

                @pl.when(j + 2 < half)
                def _():
                    pltpu.async_copy(src_ref.at[0, s, g + 2], isb, sis)

                @pl.when(j >= 2)
                def _():
                    pltpu.make_async_copy(dst_ref.at[s, g], idb.at[0],
                                          sid).wait()

                pltpu.sync_copy(buf, acc.at[idb.at[0]], add=True)

                @pl.when(j + 2 < half)
                def _():
                    pltpu.async_copy(dst_ref.at[s, g + 2], idb.at[0], sid)
                    pltpu.make_async_copy(src_ref.at[0, s, g + 2], isb,
                                          sis).wait()
                    pltpu.async_copy(table_ref.at[isb], buf, sem)
            return carry
        lax.fori_loop(0, half // 2, step, None)

        plsc.subcore_barrier()
        pltpu.sync_copy(acc.at[pl.ds(s * rpt, rpt)],
                        out_ref.at[c, pl.ds(s * rpt, rpt)])

    fn = pl.kernel(
        body,
        out_type=jax.ShapeDtypeStruct((2, MP, 128), jnp.float32),
        mesh=mesh,
        scratch_types=scratch,
        compiler_params=_SC_PARAMS,
    )
    return fn(table, src_adj, dst_slab)


def _pred_fused(table, pidx, wp2, bp2p):
    mesh = plsc.VectorSubcoreMesh(core_axis_name="c", subcore_axis_name="s")
    scratch = [
        pltpu.VMEM((PCH, 128), jnp.int32),
        pltpu.VMEM((PCH, 128), jnp.int32),
        pltpu.VMEM((128, 128), jnp.float32),
        pltpu.VMEM((128, 128), jnp.float32),
        pltpu.VMEM((128, 128), jnp.float32),
        pltpu.VMEM((128, 128), jnp.float32),
        pltpu.VMEM((128,), jnp.float32),
        pltpu.VMEM((16 * 17,), jnp.float32),
        pltpu.VMEM((PRED_HID,), jnp.float32),
        pltpu.VMEM((16,), jnp.float32),
        pltpu.SemaphoreType.DMA,
        pltpu.SemaphoreType.DMA,
        pltpu.SemaphoreType.DMA,
        pltpu.SemaphoreType.DMA,
    ]
    NK = PRED_HID // 16

    def body(table_ref, pidx_ref, wp2_ref, bp2_ref, out_ref,
             idx_a, idx_b, bufa0, bufa1, bufb0, bufb1, lbuf, tbuf,
             cwp2, cbp2, sa0, sa1, sb0, sb1):
        c = lax.axis_index("c")
        s = lax.axis_index("s")
        w = c * NS + s
        base = w * (PCH * 128)
        pltpu.sync_copy(pidx_ref.at[0, w], idx_a)
        pltpu.sync_copy(pidx_ref.at[1, w], idx_b)
        pltpu.sync_copy(wp2_ref, cwp2)
        pltpu.sync_copy(bp2_ref, cbp2)

        pltpu.async_copy(table_ref.at[idx_a.at[0]], bufa0, sa0)
        pltpu.async_copy(table_ref.at[idx_b.at[0]], bufb0, sb0)
        pltpu.async_copy(table_ref.at[idx_a.at[1]], bufa1, sa1)
        pltpu.async_copy(table_ref.at[idx_b.at[1]], bufb1, sb1)

        wp2c = [cwp2[pl.ds(k * 16, 16)] for k in range(NK)]
        bp2v = cbp2[pl.ds(0, 16)]
        lane17 = lax.iota(jnp.int32, 16) * 17

        def step(i, carry):
            for par, bufa, bufb, sa, sb in ((0, bufa0, bufb0, sa0, sb0),
                                            (1, bufa1, bufb1, sa1, sb1)):
                j = 2 * i + par
                pltpu.make_async_copy(table_ref.at[idx_a.at[j]], bufa,
                                      sa).wait()
                pltpu.make_async_copy(table_ref.at[idx_b.at[j]], bufb,
                                      sb).wait()

                def group(g, carry2):
                    def edge(ee, carry3):
                        e = g * 16 + ee
                        acc = jnp.zeros((16,), jnp.float32)
                        for k in range(NK):
                            va = bufa[e, pl.ds(k * 16, 16)]
                            vb = bufb[e, pl.ds(k * 16, 16)]
                            acc = acc + jnp.maximum(va + vb, 0.0) * wp2c[k]
                        plsc.store_scatter(tbuf, [lane17 + ee], acc)
                        return carry3
                    lax.fori_loop(0, 16, edge, None)

                    v = tbuf[pl.ds(0, 16)]
                    for l in range(1, 16):
                        v = v + tbuf[pl.ds(l * 17, 16)]
                    v = v + bp2v
                    lbuf[pl.ds(g * 16, 16)] = 1.0 / (1.0 + jnp.exp(-v))
                    return carry2
                lax.fori_loop(0, 8, group, None)

                pltpu.sync_copy(lbuf, out_ref.at[pl.ds(base + j * 128, 128)])

                @pl.when(j + 2 < PCH)
                def _():
                    pltpu.async_copy(table_ref.at[idx_a.at[j + 2]], bufa, sa)
                    pltpu.async_copy(table_ref.at[idx_b.at[j + 2]], bufb, sb)
            return carry
        lax.fori_loop(0, PCH // 2, step, None)

    fn = pl.kernel(
        body,
        out_type=jax.ShapeDtypeStruct((P_PAD,), jnp.float32),
        mesh=mesh,
        scratch_types=scratch,
        compiler_params=_SC_PARAMS,
    )
    return fn(table, pidx, wp2, bp2p)


def kernel(x, edge_index, pred_edge_index, W1l, b1l, W1r, W2l, b2l, W2r,
           W3l, b3l, W3r, Wp1, bp1, Wp2, bp2):
    ei = edge_index.astype(jnp.int32)
    pei = pred_edge_index.astype(jnp.int32)

    src = jnp.pad(ei[0], (0, E_PAD - N_EDGES))
    dst = jnp.pad(ei[1], (0, E_PAD - N_EDGES), constant_values=N_NODES)
    src_slab = src.reshape(NS, ECH, EC)
    src_adj = jnp.stack([src_slab, src_slab + MP])
    dst_slab = dst.reshape(NS, ECH, EC)

    ps = jnp.pad(pei[0], (0, P_PAD - N_PRED)).reshape(NC * NS, PCH, 128)
    pd = jnp.pad(pei[1], (0, P_PAD - N_PRED)).reshape(NC * NS, PCH, 128)
    pidx = jnp.stack([ps, pd + MP])

    cntp = _degree(dst_slab).reshape(NC * NS, MP)

    star1, xr1 = _dual_mm(x.T, W1l, W1r)
    s1 = _seg_half(star1.reshape(2 * MP, 128), src_adj, dst_slab)

    star2, xr2 = _tc_stage(s1, cntp, xr1, b1l.reshape(1, HID), W2l, W2r,
                           relu=True, split=True)
    s2 = _seg_half(star2.reshape(2 * MP, 128), src_adj, dst_slab)
    star3, xr3 = _tc_stage(s2, cntp, xr2, b2l.reshape(1, HID), W3l, W3r,
                           relu=True, split=False)
    s3p = _seg_ep(star3, src_adj, dst_slab)

    starp = _tc_final_nodes(s3p, cntp, xr3, b3l.reshape(1, OUT), Wp1,
                            bp1.reshape(1, PRED_HID))
    out = _pred_fused(starp.reshape(2 * MP, PRED_HID), pidx,
                      Wp2.reshape(PRED_HID), jnp.broadcast_to(bp2, (16,)))
    return out[:N_PRED]


# speedup vs baseline: 1.1041x; 1.0010x over previous
import jax
import jax.numpy as jnp
from jax import lax
from jax.experimental import pallas as pl
from jax.experimental.pallas import tpu as pltpu
from jax.experimental.pallas import tpu_sc as plsc

N_NODES = 10000
N_EDGES = 160000
N_PRED = 100000
IN_CH = 980
HID = 256
OUT = 64
PRED_HID = 128

NC, NS = 2, 16
MP = 10240
KP = 1024
EC = 128
ECH = 80
E_PAD = NS * ECH * EC
PCH = 26
P_PAD = NC * NS * PCH * 128
MB = 256

_SC_PARAMS = pltpu.CompilerParams(needs_layout_passes=False)


def _dual_mm(xt, Wa, Wb):
    K = xt.shape[0]
    Fo = Wa.shape[1]
    dn = (((0,), (0,)), ((), ()))

    def body(x_ref, wa_ref, wb_ref, star_ref, xr_ref):
        xb = x_ref[...]
        ol = lax.dot_general(xb, wa_ref[...], dn,
                             preferred_element_type=jnp.float32)
        star_ref[...] = jnp.stack([ol[:, :128], ol[:, 128:]])
        xr_ref[...] = lax.dot_general(xb, wb_ref[...], dn,
                                      preferred_element_type=jnp.float32)

    return pl.pallas_call(
        body,
        grid=(MP // MB,),
        in_specs=[
            pl.BlockSpec((K, MB), lambda m: (0, m)),
            pl.BlockSpec((K, Fo), lambda m: (0, 0)),
            pl.BlockSpec((K, Fo), lambda m: (0, 0)),
        ],
        out_specs=[
            pl.BlockSpec((2, MB, 128), lambda m: (0, m, 0)),
            pl.BlockSpec((MB, Fo), lambda m: (m, 0)),
        ],
        out_shape=[
            jax.ShapeDtypeStruct((2, MP, 128), jnp.float32),
            jax.ShapeDtypeStruct((MP, Fo), jnp.float32),
        ],
    )(xt, Wa, Wb)


def _tc_stage(s_star, cntp, xr, b, Wa, Wb, relu, split):
    Fin = xr.shape[1]
    Fo = Wa.shape[1]

    def body(s0_ref, s1_ref, cnt_ref, xr_ref, b_ref, wa_ref, wb_ref,
             star_ref, xro_ref):
        s = jnp.concatenate([s0_ref[0], s1_ref[0]], axis=-1)
        cnt = jnp.sum(cnt_ref[...], axis=0)[:, None]
        inv = 1.0 / jnp.maximum(cnt, 1.0)
        h = s * inv + b_ref[...] + xr_ref[...]
        if relu:
            h = jnp.maximum(h, 0.0)
        ol = jnp.dot(h, wa_ref[...], preferred_element_type=jnp.float32)
        if split:
            star_ref[...] = jnp.stack([ol[:, :128], ol[:, 128:]])
        else:
            star_ref[...] = jnp.concatenate(
                [ol, jnp.zeros((MB, 128 - Fo), jnp.float32)], axis=-1)
        xro_ref[...] = jnp.dot(h, wb_ref[...], preferred_element_type=jnp.float32)

    star_spec = (pl.BlockSpec((2, MB, 128), lambda m: (0, m, 0)) if split
                 else pl.BlockSpec((MB, 128), lambda m: (m, 0)))
    star_shape = ((2, MP, 128) if split else (MP, 128))
    return pl.pallas_call(
        body,
        grid=(MP // MB,),
        in_specs=[
            pl.BlockSpec((1, MB, 128), lambda m: (0, m, 0)),
            pl.BlockSpec((1, MB, 128), lambda m: (1, m, 0)),
            pl.BlockSpec((NC * NS, MB), lambda m: (0, m)),
            pl.BlockSpec((MB, Fin), lambda m: (m, 0)),
            pl.BlockSpec((1, Fin), lambda m: (0, 0)),
            pl.BlockSpec((Fin, Fo), lambda m: (0, 0)),
            pl.BlockSpec((Fin, Fo), lambda m: (0, 0)),
        ],
        out_specs=[
            star_spec,
            pl.BlockSpec((MB, Fo), lambda m: (m, 0)),
        ],
        out_shape=[
            jax.ShapeDtypeStruct(star_shape, jnp.float32),
            jax.ShapeDtypeStruct((MP, Fo), jnp.float32),
        ],
    )(s_star, s_star, cntp, xr, b, Wa, Wb)


def _tc_final_nodes(s3p, cntp, xr, b, Wp1, bp1):
    Fin = xr.shape[1]

    def body(s0_ref, s1_ref, cnt_ref, xr_ref, b_ref, w_ref, bp1_ref,
             star_ref):
        z = (s0_ref[0] + s1_ref[0])[:, :Fin]
        cnt = jnp.sum(cnt_ref[...], axis=0)[:, None]
        inv = 1.0 / jnp.maximum(cnt, 1.0)
        z = z * inv + b_ref[...] + xr_ref[...]
        w = w_ref[...]
        a = jnp.dot(z, w[:Fin], preferred_element_type=jnp.float32)
        a = a + bp1_ref[...]
        bz = jnp.dot(z, w[Fin:], preferred_element_type=jnp.float32)
        star_ref[...] = jnp.stack([a, bz])

    return pl.pallas_call(
        body,
        grid=(MP // MB,),
        in_specs=[
            pl.BlockSpec((1, MB, 128), lambda m: (0, m, 0)),
            pl.BlockSpec((1, MB, 128), lambda m: (1, m, 0)),
            pl.BlockSpec((NC * NS, MB), lambda m: (0, m)),
            pl.BlockSpec((MB, Fin), lambda m: (m, 0)),
            pl.BlockSpec((1, Fin), lambda m: (0, 0)),
            pl.BlockSpec((2 * Fin, PRED_HID), lambda m: (0, 0)),
            pl.BlockSpec((1, PRED_HID), lambda m: (0, 0)),
        ],
        out_specs=pl.BlockSpec((2, MB, PRED_HID), lambda m: (0, m, 0)),
        out_shape=jax.ShapeDtypeStruct((2, MP, PRED_HID), jnp.float32),
    )(s3p, s3p, cntp, xr, b, Wp1, bp1)


def _degree(dst_slab):
    half = ECH // 2
    mesh = plsc.VectorSubcoreMesh(core_axis_name="c", subcore_axis_name="s")
    scratch = [
        pltpu.VMEM((MP,), jnp.float32),
        pltpu.VMEM((1, EC), jnp.int32),
        pltpu.VMEM((1, EC), jnp.int32),
        pltpu.SemaphoreType.DMA,
        pltpu.SemaphoreType.DMA,
    ]

    def body(dst_ref, out_ref, cntp, id0, id1, sd0, sd1):
        c = lax.axis_index("c")
        s = lax.axis_index("s")
        base = c * half

        def czero(i, carry):
            cntp[pl.ds(i * 16, 16)] = jnp.zeros((16,), jnp.float32)
            return carry
        lax.fori_loop(0, MP // 16, czero, None)

        pltpu.sync_copy(dst_ref.at[s, base], id0.at[0])
        pltpu.sync_copy(dst_ref.at[s, base + 1], id1.at[0])

        ones16 = jnp.full((16,), 1.0, jnp.float32)

        def step(i, carry):
            for par, idb, sd in ((0, id0, sd0), (1, id1, sd1)):
                j = 2 * i + par

                @pl.when(j >= 2)
                def _():
                    pltpu.make_async_copy(dst_ref.at[s, base + j], idb.at[0],
                                          sd).wait()

                for k in range(EC // 16):
                    idv = idb[0, pl.ds(k * 16, 16)]
                    plsc.addupdate_scatter(cntp, [idv], ones16)

                @pl.when(j + 2 < half)
                def _():
                    pltpu.async_copy(dst_ref.at[s, base + j + 2], idb.at[0],
                                     sd)
            return carry
        lax.fori_loop(0, half // 2, step, None)

        pltpu.sync_copy(cntp, out_ref.at[c, s])

    fn = pl.kernel(body,
                   out_type=jax.ShapeDtypeStruct((NC, NS, MP), jnp.float32),
                   mesh=mesh, scratch_types=scratch,
                   compiler_params=_SC_PARAMS)
    return fn(dst_slab)


def _seg_half(table, src_adj, dst_slab):
    rpt = MP // NS

    mesh = plsc.VectorSubcoreMesh(core_axis_name="c", subcore_axis_name="s")
    scratch = [
        pltpu.VMEM_SHARED((MP, 128), jnp.float32),
        pltpu.VMEM((EC,), jnp.int32),
        pltpu.VMEM((EC,), jnp.int32),
        pltpu.VMEM((1, EC), jnp.int32),
        pltpu.VMEM((1, EC), jnp.int32),
        pltpu.VMEM((EC, 128), jnp.float32),
        pltpu.VMEM((EC, 128), jnp.float32),
        pltpu.SemaphoreType.DMA,
        pltpu.SemaphoreType.DMA,
        pltpu.SemaphoreType.DMA,
        pltpu.SemaphoreType.DMA,
        pltpu.SemaphoreType.DMA,
        pltpu.SemaphoreType.DMA,
    ]

    def body(table_ref, src_ref, dst_ref, out_ref,
             acc, is0, is1, id0, id1, buf0, buf1,
             sem0, sem1, sis0, sis1, sid0, sid1):
        c = lax.axis_index("c")
        s = lax.axis_index("s")

        def zfill(i, carry):
            for k in range(8):
                buf0[i, pl.ds(k * 16, 16)] = jnp.zeros((16,), jnp.float32)
            return carry
        lax.fori_loop(0, EC, zfill, None)
        for t in range(rpt // EC):
            pltpu.sync_copy(buf0, acc.at[pl.ds(s * rpt + t * EC, EC)])

        plsc.subcore_barrier()

        pltpu.sync_copy(src_ref.at[c, s, 0], is0)
        pltpu.sync_copy(dst_ref.at[s, 0], id0.at[0])
        pltpu.sync_copy(src_ref.at[c, s, 1], is1)
        pltpu.sync_copy(dst_ref.at[s, 1], id1.at[0])
        pltpu.async_copy(table_ref.at[is0], buf0, sem0)
        pltpu.async_copy(table_ref.at[is1], buf1, sem1)

        def step(i, carry):
            for par, isb, idb, buf, sem, sis, sid in (
                    (0, is0, id0, buf0, sem0, sis0, sid0),
                    (1, is1, id1, buf1, sem1, sis1, sid1)):
                j = 2 * i + par
                pltpu.make_async_copy(table_ref.at[isb], buf, sem).wait()

                @pl.when(j + 2 < ECH)
                def _():
                    pltpu.async_copy(src_ref.at[c, s, j + 2], isb, sis)

                @pl.when(j >= 2)
                def _():
                    pltpu.make_async_copy(dst_ref.at[s, j], idb.at[0],
                                          sid).wait()

                pltpu.sync_copy(buf, acc.at[idb.at[0]], add=True)

                @pl.when(j + 2 < ECH)
                def _():
                    pltpu.async_copy(dst_ref.at[s, j + 2], idb.at[0], sid)
                    pltpu.make_async_copy(src_ref.at[c, s, j + 2], isb,
                                          sis).wait()
                    pltpu.async_copy(table_ref.at[isb], buf, sem)
            return carry
        lax.fori_loop(0, ECH // 2, step, None)

        plsc.subcore_barrier()
        pltpu.sync_copy(acc.at[pl.ds(s * rpt, rpt)],
                        out_ref.at[c, pl.ds(s * rpt, rpt)])

    fn = pl.kernel(body,
                   out_type=jax.ShapeDtypeStruct((2, MP, 128), jnp.float32),
                   mesh=mesh, scratch_types=scratch,
                   compiler_params=_SC_PARAMS)
    return fn(table, src_adj, dst_slab)


def _seg_ep(table, src_adj, dst_slab):
    rpt = MP // NS
    half = ECH // 2

    mesh = plsc.VectorSubcoreMesh(core_axis_name="c", subcore_axis_name="s")
    scratch = [
        pltpu.VMEM_SHARED((MP, 128), jnp.float32),
        pltpu.VMEM((EC,), jnp.int32),
        pltpu.VMEM((EC,), jnp.int32),
        pltpu.VMEM((1, EC), jnp.int32),
        pltpu.VMEM((1, EC), jnp.int32),
        pltpu.VMEM((EC, 128), jnp.float32),
        pltpu.VMEM((EC, 128), jnp.float32),
        pltpu.SemaphoreType.DMA,
        pltpu.SemaphoreType.DMA,
        pltpu.SemaphoreType.DMA,
        pltpu.SemaphoreType.DMA,
        pltpu.SemaphoreType.DMA,
        pltpu.SemaphoreType.DMA,
    ]

    def body(table_ref, src_ref, dst_ref, out_ref,
             acc, is0, is1, id0, id1, buf0, buf1,
             sem0, sem1, sis0, sis1, sid0, sid1):
        c = lax.axis_index("c")
        s = lax.axis_index("s")
        base = c * half

        def zfill(i, carry):
            for k in range(8):
                buf0[i, pl.ds(k * 16, 16)] = jnp.zeros((16,), jnp.float32)
            return carry
        lax.fori_loop(0, EC, zfill, None)
        for t in range(rpt // EC):
            pltpu.sync_copy(buf0, acc.at[pl.ds(s * rpt + t * EC, EC)])

        plsc.subcore_barrier()

        pltpu.sync_copy(src_ref.at[0, s, base], is0)
        pltpu.sync_copy(dst_ref.at[s, base], id0.at[0])
        pltpu.sync_copy(src_ref.at[0, s, base + 1], is1)
        pltpu.sync_copy(dst_ref.at[s, base + 1], id1.at[0])
        pltpu.async_copy(table_ref.at[is0], buf0, sem0)
        pltpu.async_copy(table_ref.at[is1], buf1, sem1)

        def step(i, carry):
            for par, isb, idb, buf, sem, sis, sid in (
                    (0, is0, id0, buf0, sem0, sis0, sid0),
                    (1, is1, id1, buf1, sem1, sis1, sid1)):
                j = 2 * i + par
                g = base + j
                pltpu.make_async_copy(table_ref.at[isb], buf, sem).wait()

                @pl.when(j + 2 < half)
                def _():
                    pltpu.async_copy(src_ref.at[0, s, g + 2], isb, sis)

                @pl.when(j >= 2)
                def _():
                    pltpu.make_async_copy(dst_ref.at[s, g], idb.at[0],
                                          sid).wait()

                pltpu.sync_copy(buf, acc.at[idb.at[0]], add=True)

                @pl.when(j + 2 < half)
                def _():
                    pltpu.async_copy(dst_ref.at[s, g + 2], idb.at[0], sid)
                    pltpu.make_async_copy(src_ref.at[0, s, g + 2], isb,
                                          sis).wait()
                    pltpu.async_copy(table_ref.at[isb], buf, sem)
            return carry
        lax.fori_loop(0, half // 2, step, None)

        plsc.subcore_barrier()
        pltpu.sync_copy(acc.at[pl.ds(s * rpt, rpt)],
                        out_ref.at[c, pl.ds(s * rpt, rpt)])

    fn = pl.kernel(
        body,
        out_type=jax.ShapeDtypeStruct((2, MP, 128), jnp.float32),
        mesh=mesh,
        scratch_types=scratch,
        compiler_params=_SC_PARAMS,
    )
    return fn(table, src_adj, dst_slab)


def _pred_fused(table, pidx, wp2, bp2p):
    mesh = plsc.VectorSubcoreMesh(core_axis_name="c", subcore_axis_name="s")
    scratch = [
        pltpu.VMEM((PCH, 128), jnp.int32),
        pltpu.VMEM((PCH, 128), jnp.int32),
        pltpu.VMEM((128, 128), jnp.float32),
        pltpu.VMEM((128, 128), jnp.float32),
        pltpu.VMEM((128, 128), jnp.float32),
        pltpu.VMEM((128, 128), jnp.float32),
        pltpu.VMEM((PCH * 128,), jnp.float32),
        pltpu.VMEM((16 * 17,), jnp.float32),
        pltpu.VMEM((PRED_HID,), jnp.float32),
        pltpu.VMEM((16,), jnp.float32),
        pltpu.SemaphoreType.DMA,
        pltpu.SemaphoreType.DMA,
        pltpu.SemaphoreType.DMA,
        pltpu.SemaphoreType.DMA,
    ]
    NK = PRED_HID // 16

    def body(table_ref, pidx_ref, wp2_ref, bp2_ref, out_ref,
             idx_a, idx_b, bufa0, bufa1, bufb0, bufb1, lbuf, tbuf,
             cwp2, cbp2, sa0, sa1, sb0, sb1):
        c = lax.axis_index("c")
        s = lax.axis_index("s")
        w = c * NS + s
        base = w * (PCH * 128)
        pltpu.sync_copy(pidx_ref.at[0, w], idx_a)
        pltpu.sync_copy(pidx_ref.at[1, w], idx_b)
        pltpu.sync_copy(wp2_ref, cwp2)
        pltpu.sync_copy(bp2_ref, cbp2)

        pltpu.async_copy(table_ref.at[idx_a.at[0]], bufa0, sa0)
        pltpu.async_copy(table_ref.at[idx_b.at[0]], bufb0, sb0)
        pltpu.async_copy(table_ref.at[idx_a.at[1]], bufa1, sa1)
        pltpu.async_copy(table_ref.at[idx_b.at[1]], bufb1, sb1)

        wp2c = [cwp2[pl.ds(k * 16, 16)] for k in range(NK)]
        bp2v = cbp2[pl.ds(0, 16)]
        lane17 = lax.iota(jnp.int32, 16) * 17

        def step(i, carry):
            for par, bufa, bufb, sa, sb in ((0, bufa0, bufb0, sa0, sb0),
                                            (1, bufa1, bufb1, sa1, sb1)):
                j = 2 * i + par
                pltpu.make_async_copy(table_ref.at[idx_a.at[j]], bufa,
                                      sa).wait()
                pltpu.make_async_copy(table_ref.at[idx_b.at[j]], bufb,
                                      sb).wait()

                def group(g, carry2):
                    def edge(ee, carry3):
                        e = g * 16 + ee
                        acc = jnp.zeros((16,), jnp.float32)
                        for k in range(NK):
                            va = bufa[e, pl.ds(k * 16, 16)]
                            vb = bufb[e, pl.ds(k * 16, 16)]
                            acc = acc + jnp.maximum(va + vb, 0.0) * wp2c[k]
                        plsc.store_scatter(tbuf, [lane17 + ee], acc)
                        return carry3
                    lax.fori_loop(0, 16, edge, None)

                    v = tbuf[pl.ds(0, 16)]
                    for l in range(1, 16):
                        v = v + tbuf[pl.ds(l * 17, 16)]
                    v = v + bp2v
                    lbuf[pl.ds(j * 128 + g * 16, 16)] = \
                        1.0 / (1.0 + jnp.exp(-v))
                    return carry2
                lax.fori_loop(0, 8, group, None)

                @pl.when(j + 2 < PCH)
                def _():
                    pltpu.async_copy(table_ref.at[idx_a.at[j + 2]], bufa, sa)
                    pltpu.async_copy(table_ref.at[idx_b.at[j + 2]], bufb, sb)
            return carry
        lax.fori_loop(0, PCH // 2, step, None)
        pltpu.sync_copy(lbuf, out_ref.at[pl.ds(base, PCH * 128)])

    fn = pl.kernel(
        body,
        out_type=jax.ShapeDtypeStruct((P_PAD,), jnp.float32),
        mesh=mesh,
        scratch_types=scratch,
        compiler_params=_SC_PARAMS,
    )
    return fn(table, pidx, wp2, bp2p)


def kernel(x, edge_index, pred_edge_index, W1l, b1l, W1r, W2l, b2l, W2r,
           W3l, b3l, W3r, Wp1, bp1, Wp2, bp2):
    ei = edge_index.astype(jnp.int32)
    pei = pred_edge_index.astype(jnp.int32)

    src = jnp.pad(ei[0], (0, E_PAD - N_EDGES))
    dst = jnp.pad(ei[1], (0, E_PAD - N_EDGES), constant_values=N_NODES)
    src_slab = src.reshape(NS, ECH, EC)
    src_adj = jnp.stack([src_slab, src_slab + MP])
    dst_slab = dst.reshape(NS, ECH, EC)

    ps = jnp.pad(pei[0], (0, P_PAD - N_PRED)).reshape(NC * NS, PCH, 128)
    pd = jnp.pad(pei[1], (0, P_PAD - N_PRED)).reshape(NC * NS, PCH, 128)
    pidx = jnp.stack([ps, pd + MP])

    cntp = _degree(dst_slab).reshape(NC * NS, MP)

    star1, xr1 = _dual_mm(x.T, W1l, W1r)
    s1 = _seg_half(star1.reshape(2 * MP, 128), src_adj, dst_slab)

    star2, xr2 = _tc_stage(s1, cntp, xr1, b1l.reshape(1, HID), W2l, W2r,
                           relu=True, split=True)
    s2 = _seg_half(star2.reshape(2 * MP, 128), src_adj, dst_slab)
    star3, xr3 = _tc_stage(s2, cntp, xr2, b2l.reshape(1, HID), W3l, W3r,
                           relu=True, split=False)
    s3p = _seg_ep(star3, src_adj, dst_slab)

    starp = _tc_final_nodes(s3p, cntp, xr3, b3l.reshape(1, OUT), Wp1,
                            bp1.reshape(1, PRED_HID))
    out = _pred_fused(starp.reshape(2 * MP, PRED_HID), pidx,
                      Wp2.reshape(PRED_HID), jnp.broadcast_to(bp2, (16,)))
    return out[:N_PRED]
